# baseline (speedup 1.0000x reference)
import functools
import jax
import jax.numpy as jnp
from jax import lax
from jax.experimental import pallas as pl
from jax.experimental.pallas import tpu as pltpu
from jax.experimental.pallas import tpu_sc as plsc

U0 = 10.0
R = 0.2

_N = 65536
_M = 64
_NC = 2
_NS = 16
_NW = _NC * _NS
_PER_W = _N // _NW
_NV = _PER_W // 16


def _nsqrt(m):
    m = jnp.maximum(m, jnp.float32(1e-30))
    i = lax.bitcast_convert_type(m, jnp.int32)
    i = jnp.int32(0x5F3759DF) - (i >> 1)
    y = lax.bitcast_convert_type(i, jnp.float32)
    for _ in range(3):
        y = y * (jnp.float32(1.5) - jnp.float32(0.5) * m * y * y)
    return m * y


def kernel(state, B0, B1):
    bsplat = jnp.repeat(
        jnp.stack([B0[:, 0], B0[:, 1], B1[:, 0], B1[:, 1]], axis=0),
        16, axis=1).reshape(4 * _M * 16)
    xs = state[:, 0]
    ys = state[:, 1]

    mesh = plsc.VectorSubcoreMesh(core_axis_name="c", subcore_axis_name="s")

    @functools.partial(
        pl.kernel,
        mesh=mesh,
        out_type=[jax.ShapeDtypeStruct((_N,), jnp.float32)] * 2,
        scratch_types=[
            pltpu.VMEM((_PER_W,), jnp.float32),
            pltpu.VMEM((_PER_W,), jnp.float32),
            pltpu.VMEM((4 * _M * 16,), jnp.float32),
            pltpu.VMEM((_PER_W,), jnp.float32),
            pltpu.VMEM((_PER_W,), jnp.float32),
        ],
    )
    def k(xs_hbm, ys_hbm, bs_hbm, o0_hbm, o1_hbm, x_v, y_v, bs_v, o0_v, o1_v):
        wid = lax.axis_index("s") * _NC + lax.axis_index("c")
        base = wid * _PER_W
        pltpu.sync_copy(xs_hbm.at[pl.ds(base, _PER_W)], x_v)
        pltpu.sync_copy(ys_hbm.at[pl.ds(base, _PER_W)], y_v)
        pltpu.sync_copy(bs_hbm, bs_v)

        def body(i, carry):
            sl = pl.ds(i * 16, 16)
            xv = x_v[sl]
            yv = y_v[sl]

            def min_d2(xrow, yrow):
                m = None
                for j in range(_M):
                    dx = xv - bs_v[pl.ds(xrow * (_M * 16) + 16 * j, 16)]
                    dy = yv - bs_v[pl.ds(yrow * (_M * 16) + 16 * j, 16)]
                    d2 = dx * dx + dy * dy
                    m = d2 if m is None else jnp.minimum(m, d2)
                return m

            o0_v[sl] = U0 * jnp.exp(-_nsqrt(min_d2(0, 1)) / R)
            o1_v[sl] = U0 * jnp.exp(-_nsqrt(min_d2(2, 3)) / R)
            return carry

        lax.fori_loop(0, _NV, body, jnp.int32(0), unroll=2)
        pltpu.sync_copy(o0_v, o0_hbm.at[pl.ds(base, _PER_W)])
        pltpu.sync_copy(o1_v, o1_hbm.at[pl.ds(base, _PER_W)])

    o0, o1 = k(xs, ys, bsplat)
    return jnp.stack([o0, o1], axis=1)


# speedup vs baseline: 1.3478x; 1.3478x over previous
import functools
import jax
import jax.numpy as jnp
from jax import lax
from jax.experimental import pallas as pl
from jax.experimental.pallas import tpu as pltpu
from jax.experimental.pallas import tpu_sc as plsc

U0 = 10.0
R = 0.2

_N = 65536
_M = 64
_NC = 2
_NS = 16
_NW = _NC * _NS
_A = 32768
_PW = _A // _NW
_NV = _PW // 16
_BR = 128


def _nsqrt(m):
    m = jnp.maximum(m, jnp.float32(1e-30))
    i = lax.bitcast_convert_type(m, jnp.int32)
    i = jnp.int32(0x5F3759DF) - (i >> 1)
    y = lax.bitcast_convert_type(i, jnp.float32)
    for _ in range(3):
        y = y * (jnp.float32(1.5) - jnp.float32(0.5) * m * y * y)
    return m * y


def _sc_half(xs, ys, bsplat):
    mesh = plsc.VectorSubcoreMesh(core_axis_name="c", subcore_axis_name="s")

    @functools.partial(
        pl.kernel,
        mesh=mesh,
        out_type=[jax.ShapeDtypeStruct((_A,), jnp.float32)] * 2,
        scratch_types=[
            pltpu.VMEM((_PW,), jnp.float32),
            pltpu.VMEM((_PW,), jnp.float32),
            pltpu.VMEM((4 * _M * 16,), jnp.float32),
            pltpu.VMEM((_PW,), jnp.float32),
            pltpu.VMEM((_PW,), jnp.float32),
        ],
    )
    def k(xs_hbm, ys_hbm, bs_hbm, o0_hbm, o1_hbm, x_v, y_v, bs_v, o0_v, o1_v):
        wid = lax.axis_index("s") * _NC + lax.axis_index("c")
        base = wid * _PW
        pltpu.sync_copy(xs_hbm.at[pl.ds(base, _PW)], x_v)
        pltpu.sync_copy(ys_hbm.at[pl.ds(base, _PW)], y_v)
        pltpu.sync_copy(bs_hbm, bs_v)

        def body(i, carry):
            sl = pl.ds(i * 16, 16)
            xv = x_v[sl]
            yv = y_v[sl]

            def min_d2(xrow, yrow):
                m = None
                for j in range(_M):
                    dx = xv - bs_v[pl.ds(xrow * (_M * 16) + 16 * j, 16)]
                    dy = yv - bs_v[pl.ds(yrow * (_M * 16) + 16 * j, 16)]
                    d2 = dx * dx + dy * dy
                    m = d2 if m is None else jnp.minimum(m, d2)
                return m

            o0_v[sl] = U0 * jnp.exp(-_nsqrt(min_d2(0, 1)) / R)
            o1_v[sl] = U0 * jnp.exp(-_nsqrt(min_d2(2, 3)) / R)
            return carry

        lax.fori_loop(0, _NV, body, jnp.int32(0), unroll=2)
        pltpu.sync_copy(o0_v, o0_hbm.at[pl.ds(base, _PW)])
        pltpu.sync_copy(o1_v, o1_hbm.at[pl.ds(base, _PW)])

    return k(xs, ys, bsplat)


def _tc_kernel(b0_ref, b1_ref, x_ref, y_ref, o0_ref, o1_ref):
    x = x_ref[...]
    y = y_ref[...]

    def min_d2(b_ref):
        m = None
        for j in range(_M):
            dx = x - b_ref[j, 0]
            dy = y - b_ref[j, 1]
            d2 = dx * dx + dy * dy
            m = d2 if m is None else jnp.minimum(m, d2)
        return m

    o0_ref[...] = U0 * jnp.exp(-jnp.sqrt(min_d2(b0_ref)) / R)
    o1_ref[...] = U0 * jnp.exp(-jnp.sqrt(min_d2(b1_ref)) / R)


def _tc_half(x2d, y2d, B0, B1):
    rows = (_N - _A) // 128
    off = _A // (128 * _BR)
    plane_in = pl.BlockSpec((_BR, 128), lambda i: (i + off, 0))
    plane_out = pl.BlockSpec((_BR, 128), lambda i: (i, 0))
    smem = pl.BlockSpec(memory_space=pltpu.SMEM)
    return pl.pallas_call(
        _tc_kernel,
        grid=(rows // _BR,),
        in_specs=[smem, smem, plane_in, plane_in],
        out_specs=[plane_out, plane_out],
        out_shape=[jax.ShapeDtypeStruct((rows, 128), jnp.float32)] * 2,
    )(B0, B1, x2d, y2d)


def kernel(state, B0, B1):
    bsplat = jnp.repeat(
        jnp.stack([B0[:, 0], B0[:, 1], B1[:, 0], B1[:, 1]], axis=0),
        16, axis=1).reshape(4 * _M * 16)
    xs = state[:, 0]
    ys = state[:, 1]

    s0, s1 = _sc_half(xs, ys, bsplat)
    t0, t1 = _tc_half(xs.reshape(512, 128), ys.reshape(512, 128), B0, B1)
    o0 = jnp.concatenate([s0, t0.reshape(-1)])
    o1 = jnp.concatenate([s1, t1.reshape(-1)])
    return jnp.stack([o0, o1], axis=1)


# speedup vs baseline: 1.6945x; 1.2572x over previous
import functools
import jax
import jax.numpy as jnp
from jax import lax
from jax.experimental import pallas as pl
from jax.experimental.pallas import tpu as pltpu
from jax.experimental.pallas import tpu_sc as plsc

U0 = 10.0
R = 0.2

_N = 65536
_M = 64
_NC = 2
_NS = 16
_NW = _NC * _NS
_A = 16384
_PW = _A // _NW
_NV = _PW // 16
_BR = 128


def _nsqrt(m):
    m = jnp.maximum(m, jnp.float32(1e-30))
    i = lax.bitcast_convert_type(m, jnp.int32)
    i = jnp.int32(0x5F3759DF) - (i >> 1)
    y = lax.bitcast_convert_type(i, jnp.float32)
    for _ in range(3):
        y = y * (jnp.float32(1.5) - jnp.float32(0.5) * m * y * y)
    return m * y


def _sc_half(xs, ys, bsplat):
    mesh = plsc.VectorSubcoreMesh(core_axis_name="c", subcore_axis_name="s")

    @functools.partial(
        pl.kernel,
        mesh=mesh,
        out_type=[jax.ShapeDtypeStruct((_A,), jnp.float32)] * 2,
        scratch_types=[
            pltpu.VMEM((_PW,), jnp.float32),
            pltpu.VMEM((_PW,), jnp.float32),
            pltpu.VMEM((4 * _M * 16,), jnp.float32),
            pltpu.VMEM((_PW,), jnp.float32),
            pltpu.VMEM((_PW,), jnp.float32),
        ],
    )
    def k(xs_hbm, ys_hbm, bs_hbm, o0_hbm, o1_hbm, x_v, y_v, bs_v, o0_v, o1_v):
        wid = lax.axis_index("s") * _NC + lax.axis_index("c")
        base = wid * _PW
        pltpu.sync_copy(xs_hbm.at[pl.ds(base, _PW)], x_v)
        pltpu.sync_copy(ys_hbm.at[pl.ds(base, _PW)], y_v)
        pltpu.sync_copy(bs_hbm, bs_v)

        def body(i, carry):
            sl = pl.ds(i * 16, 16)
            xv = x_v[sl]
            yv = y_v[sl]

            def min_d2(xrow, yrow):
                m = None
                for j in range(_M):
                    dx = xv - bs_v[pl.ds(xrow * (_M * 16) + 16 * j, 16)]
                    dy = yv - bs_v[pl.ds(yrow * (_M * 16) + 16 * j, 16)]
                    d2 = dx * dx + dy * dy
                    m = d2 if m is None else jnp.minimum(m, d2)
                return m

            o0_v[sl] = U0 * jnp.exp(-_nsqrt(min_d2(0, 1)) / R)
            o1_v[sl] = U0 * jnp.exp(-_nsqrt(min_d2(2, 3)) / R)
            return carry

        lax.fori_loop(0, _NV, body, jnp.int32(0), unroll=2)
        pltpu.sync_copy(o0_v, o0_hbm.at[pl.ds(base, _PW)])
        pltpu.sync_copy(o1_v, o1_hbm.at[pl.ds(base, _PW)])

    return k(xs, ys, bsplat)


def _tc_kernel(b0_ref, b1_ref, x_ref, y_ref, o0_ref, o1_ref):
    x = x_ref[...]
    y = y_ref[...]

    def min_d2(b_ref):
        m = None
        for j in range(_M):
            dx = x - b_ref[j, 0]
            dy = y - b_ref[j, 1]
            d2 = dx * dx + dy * dy
            m = d2 if m is None else jnp.minimum(m, d2)
        return m

    o0_ref[...] = U0 * jnp.exp(-jnp.sqrt(min_d2(b0_ref)) / R)
    o1_ref[...] = U0 * jnp.exp(-jnp.sqrt(min_d2(b1_ref)) / R)


def _tc_half(x2d, y2d, B0, B1):
    rows = (_N - _A) // 128
    off = _A // (128 * _BR)
    plane_in = pl.BlockSpec((_BR, 128), lambda i: (i + off, 0))
    plane_out = pl.BlockSpec((_BR, 128), lambda i: (i, 0))
    smem = pl.BlockSpec(memory_space=pltpu.SMEM)
    return pl.pallas_call(
        _tc_kernel,
        grid=(rows // _BR,),
        in_specs=[smem, smem, plane_in, plane_in],
        out_specs=[plane_out, plane_out],
        out_shape=[jax.ShapeDtypeStruct((rows, 128), jnp.float32)] * 2,
    )(B0, B1, x2d, y2d)


def kernel(state, B0, B1):
    bsplat = jnp.repeat(
        jnp.stack([B0[:, 0], B0[:, 1], B1[:, 0], B1[:, 1]], axis=0),
        16, axis=1).reshape(4 * _M * 16)
    xs = state[:, 0]
    ys = state[:, 1]

    s0, s1 = _sc_half(xs, ys, bsplat)
    t0, t1 = _tc_half(xs.reshape(512, 128), ys.reshape(512, 128), B0, B1)
    o0 = jnp.concatenate([s0, t0.reshape(-1)])
    o1 = jnp.concatenate([s1, t1.reshape(-1)])
    return jnp.stack([o0, o1], axis=1)
